# Initial kernel scaffold; baseline (speedup 1.0000x reference)
import jax
import jax.numpy as jnp
from jax.experimental import pallas as pl


def kernel(inputs, w_B, w_AB):
    raise NotImplementedError("write your pallas kernel here")


# speedup vs baseline: 5.3816x; 5.3816x over previous
import functools

import jax
import jax.numpy as jnp
from jax import lax
from jax.experimental import pallas as pl
from jax.experimental.pallas import tpu as pltpu
from jax.experimental.pallas import tpu_sc as plsc

_N = 1000
_B = 16384

_NC = 2
_NS = 16
_NW = _NC * _NS
_CHUNK = _B // _NW
_L = 16
_GCHUNK = 128


def _comb_body(w_ref, wb_ref, comb_ref):
    w = w_ref[...]
    m = jnp.max(w, axis=1, keepdims=True)
    lse = jnp.log(jnp.sum(jnp.exp(w - m), axis=1, keepdims=True)) + m
    wb = wb_ref[...]
    mb = jnp.max(wb)
    lseb = jnp.log(jnp.sum(jnp.exp(wb - mb))) + mb
    comb_ref[...] = wb - lse - lseb


def _sc_body(a_hbm, b_hbm, tab_hbm, comb_hbm, out_hbm,
             a_v, b_v, idx_v, g_v, c_v, sem):
    wid = lax.axis_index("s") * _NC + lax.axis_index("c")
    base = wid * _CHUNK
    pltpu.sync_copy(a_hbm.at[pl.ds(base, _CHUNK)], a_v)
    pltpu.sync_copy(b_hbm.at[pl.ds(base, _CHUNK)], b_v)
    for i in range(_CHUNK // _L):
        s = pl.ds(i * _L, _L)
        idx_v[s] = b_v[s] * _N + a_v[s]
    cps = []
    for j in range(_CHUNK // _GCHUNK):
        s = pl.ds(j * _GCHUNK, _GCHUNK)
        cps.append(pltpu.async_copy(tab_hbm.at[idx_v.at[s]], g_v.at[s], sem))
        cps.append(pltpu.async_copy(comb_hbm.at[b_v.at[s]], c_v.at[s], sem))
    for c in cps:
        c.wait()
    for i in range(_CHUNK // _L):
        s = pl.ds(i * _L, _L)
        g_v[s] = g_v[s] + c_v[s]
    pltpu.sync_copy(g_v, out_hbm.at[pl.ds(base, _CHUNK)])


def kernel(inputs, w_B, w_AB):
    a = inputs[:, 0].astype(jnp.int32)
    b = inputs[:, 1].astype(jnp.int32)
    comb = pl.pallas_call(
        _comb_body,
        out_shape=jax.ShapeDtypeStruct((_N, 1), jnp.float32),
    )(w_AB, w_B.reshape(_N, 1))
    mesh = plsc.VectorSubcoreMesh(core_axis_name="c", subcore_axis_name="s")
    sc = pl.kernel(
        _sc_body,
        mesh=mesh,
        out_type=jax.ShapeDtypeStruct((_B,), jnp.float32),
        scratch_types=[
            pltpu.VMEM((_CHUNK,), jnp.int32),
            pltpu.VMEM((_CHUNK,), jnp.int32),
            pltpu.VMEM((_CHUNK,), jnp.int32),
            pltpu.VMEM((_CHUNK,), jnp.float32),
            pltpu.VMEM((_CHUNK,), jnp.float32),
            pltpu.SemaphoreType.DMA,
        ],
    )
    return sc(a, b, w_AB.reshape(-1), comb.reshape(-1))


# speedup vs baseline: 6.6459x; 1.2349x over previous
import jax
import jax.numpy as jnp
from jax import lax
from jax.experimental import pallas as pl
from jax.experimental.pallas import tpu as pltpu
from jax.experimental.pallas import tpu_sc as plsc

_N = 1000
_B = 16384

_NC = 2
_NS = 16
_NW = _NC * _NS
_CHUNK = _B // _NW
_L = 16
_GCHUNK = 128
_ROWS = 200


def _tab_body(w_ref, wbfull_ref, wb_ref, out_ref):
    w = w_ref[...]
    m = jnp.max(w, axis=1, keepdims=True)
    lse = jnp.log(jnp.sum(jnp.exp(w - m), axis=1, keepdims=True)) + m
    wbf = wbfull_ref[...]
    mb = jnp.max(wbf)
    lseb = jnp.log(jnp.sum(jnp.exp(wbf - mb))) + mb
    out_ref[...] = w - lse + wb_ref[...] - lseb


def _sc_body(a_hbm, b_hbm, tab_hbm, out_hbm, a_v, b_v, idx_v, g_v, sem):
    wid = lax.axis_index("s") * _NC + lax.axis_index("c")
    base = wid * _CHUNK
    cp_a = pltpu.async_copy(a_hbm.at[pl.ds(base, _CHUNK)], a_v, sem)
    cp_b = pltpu.async_copy(b_hbm.at[pl.ds(base, _CHUNK)], b_v, sem)
    cp_a.wait()
    cp_b.wait()
    for i in range(_CHUNK // _L):
        s = pl.ds(i * _L, _L)
        idx_v[s] = b_v[s] * _N + a_v[s]
    cps = []
    for j in range(_CHUNK // _GCHUNK):
        s = pl.ds(j * _GCHUNK, _GCHUNK)
        cps.append(pltpu.async_copy(tab_hbm.at[idx_v.at[s]], g_v.at[s], sem))
    for c in cps:
        c.wait()
    pltpu.sync_copy(g_v, out_hbm.at[pl.ds(base, _CHUNK)])


def kernel(inputs, w_B, w_AB):
    a = inputs[:, 0].astype(jnp.int32)
    b = inputs[:, 1].astype(jnp.int32)
    wb2 = w_B.reshape(_N, 1)
    table2 = pl.pallas_call(
        _tab_body,
        grid=(_N // _ROWS,),
        in_specs=[
            pl.BlockSpec((_ROWS, _N), lambda i: (i, 0)),
            pl.BlockSpec((_N, 1), lambda i: (0, 0)),
            pl.BlockSpec((_ROWS, 1), lambda i: (i, 0)),
        ],
        out_specs=pl.BlockSpec((_ROWS, _N), lambda i: (i, 0)),
        out_shape=jax.ShapeDtypeStruct((_N, _N), jnp.float32),
    )(w_AB, wb2, wb2)
    mesh = plsc.VectorSubcoreMesh(core_axis_name="c", subcore_axis_name="s")
    sc = pl.kernel(
        _sc_body,
        mesh=mesh,
        out_type=jax.ShapeDtypeStruct((_B,), jnp.float32),
        scratch_types=[
            pltpu.VMEM((_CHUNK,), jnp.int32),
            pltpu.VMEM((_CHUNK,), jnp.int32),
            pltpu.VMEM((_CHUNK,), jnp.int32),
            pltpu.VMEM((_CHUNK,), jnp.float32),
            pltpu.SemaphoreType.DMA,
        ],
    )
    return sc(a, b, table2.reshape(-1))


# speedup vs baseline: 8.4392x; 1.2699x over previous
import jax
import jax.numpy as jnp
from jax import lax
from jax.experimental import pallas as pl
from jax.experimental.pallas import tpu as pltpu
from jax.experimental.pallas import tpu_sc as plsc

_N = 1000
_B = 16384

_NC = 2
_NS = 16
_NW = _NC * _NS
_CHUNK = _B // _NW
_L = 16
_GCHUNK = 128
_ROWS = 128


def _tab_body(w_ref, wb_ref, out_ref):
    w = w_ref[...]
    lane = lax.broadcasted_iota(jnp.int32, (_ROWS, 1024), 1)
    wm = jnp.where(lane < _N, w, -jnp.inf)
    m = jnp.max(wm, axis=1, keepdims=True)
    lse = jnp.log(jnp.sum(jnp.exp(wm - m), axis=1, keepdims=True)) + m
    wb = wb_ref[...]
    lane1 = lax.broadcasted_iota(jnp.int32, (1024,), 0)
    wbm = jnp.where(lane1 < _N, wb, -jnp.inf)
    mb = jnp.max(wbm)
    lseb = jnp.log(jnp.sum(jnp.exp(wbm - mb))) + mb
    i = pl.program_id(0)
    wb_row = wb_ref[pl.ds(i * _ROWS, _ROWS)].reshape(1, _ROWS)
    wb_col = wb_row.T
    corr = lse - wb_col + lseb
    for k in range(8):
        out_ref[pl.ds(_ROWS * k, _ROWS), :] = w[:, 128 * k:128 * (k + 1)] - corr


def _sc_body(a_hbm, b_hbm, tab_hbm, out_hbm, a_v, b_v, idx_v, g_v, sem):
    wid = lax.axis_index("s") * _NC + lax.axis_index("c")
    base = wid * _CHUNK
    cp_a = pltpu.async_copy(a_hbm.at[pl.ds(base, _CHUNK)], a_v, sem)
    cp_b = pltpu.async_copy(b_hbm.at[pl.ds(base, _CHUNK)], b_v, sem)
    cp_a.wait()
    cp_b.wait()
    for i in range(_CHUNK // _L):
        s = pl.ds(i * _L, _L)
        av = a_v[s]
        bv = b_v[s]
        idx_v[s] = ((bv >> 7) * 131072 + (av >> 7) * 16384
                    + (bv & 127) * 128 + (av & 127))
    cps = []
    for j in range(_CHUNK // _GCHUNK):
        s = pl.ds(j * _GCHUNK, _GCHUNK)
        cps.append(pltpu.async_copy(tab_hbm.at[idx_v.at[s]], g_v.at[s], sem))
    for c in cps:
        c.wait()
    pltpu.sync_copy(g_v, out_hbm.at[pl.ds(base, _CHUNK)])


def kernel(inputs, w_B, w_AB):
    a = inputs[:, 0].astype(jnp.int32)
    b = inputs[:, 1].astype(jnp.int32)
    table = pl.pallas_call(
        _tab_body,
        grid=(8,),
        in_specs=[
            pl.BlockSpec((_ROWS, 1024), lambda i: (i, 0)),
            pl.BlockSpec((1024,), lambda i: (0,)),
        ],
        out_specs=pl.BlockSpec((8 * _ROWS, 128), lambda i: (i, 0)),
        out_shape=jax.ShapeDtypeStruct((64 * _ROWS, 128), jnp.float32),
    )(w_AB, w_B)
    mesh = plsc.VectorSubcoreMesh(core_axis_name="c", subcore_axis_name="s")
    sc = pl.kernel(
        _sc_body,
        mesh=mesh,
        out_type=jax.ShapeDtypeStruct((_B,), jnp.float32),
        scratch_types=[
            pltpu.VMEM((_CHUNK,), jnp.int32),
            pltpu.VMEM((_CHUNK,), jnp.int32),
            pltpu.VMEM((_CHUNK,), jnp.int32),
            pltpu.VMEM((_CHUNK,), jnp.float32),
            pltpu.SemaphoreType.DMA,
        ],
    )
    return sc(a, b, table.reshape(-1))
